# Initial kernel scaffold; baseline (speedup 1.0000x reference)
import jax
import jax.numpy as jnp
from jax.experimental import pallas as pl


def kernel(in_features, Wg, bg, Wu, bu, Wd, bd, Wfc, bfc):
    raise NotImplementedError("write your pallas kernel here")


# speedup vs baseline: 1.2987x; 1.2987x over previous
import functools

import jax
import jax.numpy as jnp
from jax.experimental import pallas as pl
from jax.experimental.pallas import tpu as pltpu

_E = 8
_D = 768
_FF = 3072
_T = 2048
_BF = 512
_NF = _FF // _BF


def _moe_body(x_ref, wfc_ref, bfc_ref, wg_ref, bg_ref, wu_ref, bu_ref,
              wd_ref, bd_ref, out_ref, xbf_ref, rout_ref):
    e = pl.program_id(0)
    f = pl.program_id(1)
    step = e * pl.num_programs(1) + f

    @pl.when(step == 0)
    def _init():
        x = x_ref[...]
        xbf_ref[...] = x.astype(jnp.bfloat16)
        logits = jnp.dot(x.astype(jnp.bfloat16),
                         wfc_ref[...].astype(jnp.bfloat16),
                         preferred_element_type=jnp.float32) + bfc_ref[...]
        s = jax.nn.softmax(logits, axis=-1)
        s = jax.nn.softmax(s, axis=-1)
        rout_ref[...] = jax.nn.softmax(s, axis=-1)
        out_ref[...] = jnp.zeros_like(out_ref)

    xb = xbf_ref[...]
    wg = wg_ref[0].astype(jnp.bfloat16)
    wu = wu_ref[0].astype(jnp.bfloat16)
    g = jnp.dot(xb, wg, preferred_element_type=jnp.float32) + bg_ref[0]
    u = jnp.dot(xb, wu, preferred_element_type=jnp.float32) + bu_ref[0]

    onehot = (jax.lax.broadcasted_iota(jnp.int32, (1, _E), 1) == e)
    r = jnp.sum(rout_ref[...] * onehot.astype(jnp.float32), axis=1,
                keepdims=True)

    h = (jnp.maximum(g, 0.0) * u * r).astype(jnp.bfloat16)
    wd = wd_ref[0].astype(jnp.bfloat16)
    out_ref[...] += jnp.dot(h, wd, preferred_element_type=jnp.float32)

    @pl.when(f == 0)
    def _bias_d():
        out_ref[...] += r * bd_ref[0]


@jax.jit
def _moe_fused(x, wg, bg, wu, bu, wd, bd, wfc, bfc):
    grid = (_E, _NF)
    return pl.pallas_call(
        _moe_body,
        grid=grid,
        in_specs=[
            pl.BlockSpec((_T, _D), lambda e, f: (0, 0)),
            pl.BlockSpec((_D, _E), lambda e, f: (0, 0)),
            pl.BlockSpec((1, _E), lambda e, f: (0, 0)),
            pl.BlockSpec((1, _D, _BF), lambda e, f: (e, 0, f)),
            pl.BlockSpec((1, 1, _BF), lambda e, f: (e, 0, f)),
            pl.BlockSpec((1, _D, _BF), lambda e, f: (e, 0, f)),
            pl.BlockSpec((1, 1, _BF), lambda e, f: (e, 0, f)),
            pl.BlockSpec((1, _BF, _D), lambda e, f: (e, f, 0)),
            pl.BlockSpec((1, 1, _D), lambda e, f: (e, 0, 0)),
        ],
        out_specs=pl.BlockSpec((_T, _D), lambda e, f: (0, 0)),
        out_shape=jax.ShapeDtypeStruct((_T, _D), jnp.float32),
        scratch_shapes=[
            pltpu.VMEM((_T, _D), jnp.bfloat16),
            pltpu.VMEM((_T, _E), jnp.float32),
        ],
        compiler_params=pltpu.CompilerParams(
            dimension_semantics=("arbitrary", "arbitrary"),
        ),
    )(x, wfc, bfc.reshape(1, _E), wg, bg.reshape(_E, 1, _FF),
      wu, bu.reshape(_E, 1, _FF), wd, bd.reshape(_E, 1, _D))


def kernel(in_features, Wg, bg, Wu, bu, Wd, bd, Wfc, bfc):
    return _moe_fused(in_features, Wg, bg, Wu, bu, Wd, bd, Wfc, bfc)
